# baseline (speedup 1.0000x reference)
import functools
import math

import jax
import jax.numpy as jnp
from jax import lax
from jax.experimental import pallas as pl
from jax.experimental.pallas import tpu as pltpu
from jax.experimental.pallas import tpu_sc as plsc

S = 64.0
M = 0.5
COS_M = math.cos(M)
SIN_M = math.sin(M)

_CH = 20000
_UNROLL = 10


def _newton_sqrt(x):
    y = 0.5 * (x + 1.0)
    for _ in range(22):
        y = 0.5 * (y + x / y)
    return y


def _make_sc_kernel(B, C):
    info = plsc.get_sparse_core_info()
    NC, NS = info.num_cores, info.num_subcores
    NW = NC * NS
    rows_pw = B // NW
    elems_pw = rows_pw * C
    nch = elems_pw // _CH
    assert elems_pw % _CH == 0 and nch % 2 == 0 and _CH % (16 * _UNROLL) == 0
    n_vec = _CH // 16

    mesh = plsc.VectorSubcoreMesh(core_axis_name="c", subcore_axis_name="s")

    @functools.partial(
        pl.kernel,
        mesh=mesh,
        out_type=jax.ShapeDtypeStruct((B * C,), jnp.float32),
        scratch_types=[
            pltpu.VMEM((_CH,), jnp.float32),
            pltpu.VMEM((_CH,), jnp.float32),
            pltpu.VMEM((_CH,), jnp.float32),
            pltpu.VMEM((_CH,), jnp.float32),
            pltpu.VMEM((rows_pw,), jnp.int32),
            pltpu.VMEM((rows_pw,), jnp.int32),
            pltpu.VMEM((rows_pw,), jnp.float32),
            pltpu.SemaphoreType.DMA,
            pltpu.SemaphoreType.DMA,
            pltpu.SemaphoreType.DMA,
            pltpu.SemaphoreType.DMA,
            pltpu.SemaphoreType.DMA,
        ],
    )
    def sc_kernel(cos_hbm, lab_hbm, out_hbm,
                  ibuf0, ibuf1, obuf0, obuf1, lbuf, fbuf, vbuf,
                  isem0, isem1, osem0, osem1, gsem):
        wid = lax.axis_index("s") * NC + lax.axis_index("c")
        base = wid * elems_pw
        ibufs = (ibuf0, ibuf1)
        obufs = (obuf0, obuf1)
        isems = (isem0, isem1)
        osems = (osem0, osem1)

        pltpu.async_copy(cos_hbm.at[pl.ds(base, _CH)], ibuf0, isem0)
        pltpu.async_copy(cos_hbm.at[pl.ds(base + _CH, _CH)], ibuf1, isem1)

        def round_(p, _):
            for b in range(2):
                k = p * 2 + b
                off = base + k * _CH
                ib, ob = ibufs[b], obufs[b]
                pltpu.make_async_copy(cos_hbm.at[pl.ds(base, _CH)], ib,
                                      isems[b]).wait()
                @pl.when(k >= 2)
                def _():
                    pltpu.make_async_copy(ob, out_hbm.at[pl.ds(base, _CH)],
                                          osems[b]).wait()

                def scale_body(i, _):
                    for u in range(_UNROLL):
                        s = i * (_UNROLL * 16) + u * 16
                        ob[pl.ds(s, 16)] = ib[pl.ds(s, 16)] * S
                    return 0

                lax.fori_loop(0, n_vec // _UNROLL, scale_body, 0,
                              unroll=False)
                @pl.when(k + 2 < nch)
                def _():
                    pltpu.async_copy(
                        cos_hbm.at[pl.ds(off + 2 * _CH, _CH)], ib, isems[b])
                pltpu.async_copy(ob, out_hbm.at[pl.ds(off, _CH)], osems[b])
            return 0

        lax.fori_loop(0, nch // 2, round_, 0, unroll=False)
        pltpu.make_async_copy(obuf0, out_hbm.at[pl.ds(base, _CH)], osem0).wait()
        pltpu.make_async_copy(obuf1, out_hbm.at[pl.ds(base, _CH)], osem1).wait()

        row0 = wid * rows_pw
        pltpu.sync_copy(lab_hbm.at[pl.ds(row0, rows_pw)], lbuf)
        for t in range(rows_pw // 16):
            lab16 = lbuf[pl.ds(t * 16, 16)]
            rows = row0 + t * 16 + lax.iota(jnp.int32, 16)
            fbuf[pl.ds(t * 16, 16)] = rows * C + lab16
        pltpu.async_copy(cos_hbm.at[fbuf], vbuf, gsem).wait()
        for t in range(rows_pw // 16):
            c = vbuf[pl.ds(t * 16, 16)]
            root = _newton_sqrt(jnp.maximum(1.0 - c * c, 1e-30))
            vbuf[pl.ds(t * 16, 16)] = (c * COS_M - root * SIN_M) * S
        pltpu.async_copy(vbuf, out_hbm.at[fbuf], gsem).wait()

    return sc_kernel


def kernel(cosine, label):
    B, C = cosine.shape
    out_flat = _make_sc_kernel(B, C)(cosine.reshape(B * C), label)
    return out_flat.reshape(B, C)


# speedup vs baseline: 2.7651x; 2.7651x over previous
import functools
import math

import jax
import jax.numpy as jnp
from jax.experimental import pallas as pl
from jax.experimental.pallas import tpu as pltpu

S = 64.0
M = 0.5
COS_M = math.cos(M)
SIN_M = math.sin(M)

_BR = 16


def _arcface_block(label_ref, cos_ref, out_ref):
    i = pl.program_id(0)
    out_ref[...] = cos_ref[...] * S
    lane = jax.lax.broadcasted_iota(jnp.int32, (1, 128), 1)
    for r in range(_BR):
        col = label_ref[i * _BR + r]
        off = jax.lax.rem(col, 128)
        base = pl.multiple_of(col - off, 128)
        c = cos_ref[pl.ds(r, 1), pl.ds(base, 128)]
        penal = (c * COS_M - jnp.sqrt(jnp.maximum(1.0 - c * c, 0.0)) * SIN_M) * S
        out_ref[pl.ds(r, 1), pl.ds(base, 128)] = jnp.where(lane == off, penal, c * S)


def kernel(cosine, label):
    B, C = cosine.shape
    grid_spec = pltpu.PrefetchScalarGridSpec(
        num_scalar_prefetch=1,
        grid=(pl.cdiv(B, _BR),),
        in_specs=[pl.BlockSpec((_BR, C), lambda i, lab: (i, 0))],
        out_specs=pl.BlockSpec((_BR, C), lambda i, lab: (i, 0)),
    )
    return pl.pallas_call(
        _arcface_block,
        grid_spec=grid_spec,
        out_shape=jax.ShapeDtypeStruct((B, C), cosine.dtype),
    )(label, cosine)


# speedup vs baseline: 2.7666x; 1.0006x over previous
import functools
import math

import jax
import jax.numpy as jnp
from jax.experimental import pallas as pl
from jax.experimental.pallas import tpu as pltpu

S = 64.0
M = 0.5
COS_M = math.cos(M)
SIN_M = math.sin(M)

_BR = 32


def _arcface_block(label_ref, cos_ref, out_ref):
    i = pl.program_id(0)
    out_ref[...] = cos_ref[...] * S
    lane = jax.lax.broadcasted_iota(jnp.int32, (1, 128), 1)
    for r in range(_BR):
        col = label_ref[i * _BR + r]
        off = jax.lax.rem(col, 128)
        base = pl.multiple_of(col - off, 128)
        c = cos_ref[pl.ds(r, 1), pl.ds(base, 128)]
        penal = (c * COS_M - jnp.sqrt(jnp.maximum(1.0 - c * c, 0.0)) * SIN_M) * S
        out_ref[pl.ds(r, 1), pl.ds(base, 128)] = jnp.where(lane == off, penal, c * S)


def kernel(cosine, label):
    B, C = cosine.shape
    grid_spec = pltpu.PrefetchScalarGridSpec(
        num_scalar_prefetch=1,
        grid=(pl.cdiv(B, _BR),),
        in_specs=[pl.BlockSpec((_BR, C), lambda i, lab: (i, 0))],
        out_specs=pl.BlockSpec((_BR, C), lambda i, lab: (i, 0)),
    )
    return pl.pallas_call(
        _arcface_block,
        grid_spec=grid_spec,
        out_shape=jax.ShapeDtypeStruct((B, C), cosine.dtype),
    )(label, cosine)


# speedup vs baseline: 2.7689x; 1.0008x over previous
import functools
import math

import jax
import jax.numpy as jnp
from jax.experimental import pallas as pl
from jax.experimental.pallas import tpu as pltpu

S = 64.0
M = 0.5
COS_M = math.cos(M)
SIN_M = math.sin(M)

_BR = 32


def _penal(c):
    return (c * COS_M - jnp.sqrt(jnp.maximum(1.0 - c * c, 0.0)) * SIN_M) * S


def _arcface_block(C, label_ref, cos_ref, out_ref):
    i = pl.program_id(0)
    out_ref[...] = cos_ref[...] * S
    C_al = (C // 128) * 128
    lane = jax.lax.broadcasted_iota(jnp.int32, (1, 128), 1)
    tail = C - C_al
    lane_t = jax.lax.broadcasted_iota(jnp.int32, (1, tail), 1) if tail else None
    for r in range(_BR):
        col = label_ref[i * _BR + r]

        @pl.when(col < C_al)
        def _():
            off = jax.lax.rem(col, 128)
            base = pl.multiple_of(col - off, 128)
            c = cos_ref[pl.ds(r, 1), pl.ds(base, 128)]
            out_ref[pl.ds(r, 1), pl.ds(base, 128)] = jnp.where(
                lane == off, _penal(c), c * S)

        if tail:
            @pl.when(col >= C_al)
            def _():
                c = cos_ref[pl.ds(r, 1), pl.ds(C_al, tail)]
                out_ref[pl.ds(r, 1), pl.ds(C_al, tail)] = jnp.where(
                    lane_t == col - C_al, _penal(c), c * S)


def kernel(cosine, label):
    B, C = cosine.shape
    grid_spec = pltpu.PrefetchScalarGridSpec(
        num_scalar_prefetch=1,
        grid=(pl.cdiv(B, _BR),),
        in_specs=[pl.BlockSpec((_BR, C), lambda i, lab: (i, 0))],
        out_specs=pl.BlockSpec((_BR, C), lambda i, lab: (i, 0)),
    )
    return pl.pallas_call(
        functools.partial(_arcface_block, C),
        grid_spec=grid_spec,
        out_shape=jax.ShapeDtypeStruct((B, C), cosine.dtype),
    )(label, cosine)
